# baseline (speedup 1.0000x reference)
import functools

import jax
import jax.numpy as jnp
from jax import lax
from jax.experimental import pallas as pl
from jax.experimental.pallas import tpu as pltpu
from jax.experimental.pallas import tpu_sc as plsc

D0, D1, D2 = 5, 6, 2
EMB = 16
NCODES = D0 * D1 * D2
NC, NS, LANES = 2, 16, 16
NW = NC * NS
CHUNK = 2000
GROW = 80
NG = CHUNK // GROW
TC_BLK = 5120


def _codes_body(attr_ref, code_ref):
    x = attr_ref[...]
    code_ref[...] = (x[:, 0] * D1 + x[:, 1]) * D2 + x[:, 2]


def _sc_body(codes_hbm, w0_hbm, w1_hbm, w2_hbm, out_hbm,
             w0_v, w1_v, w2_v, lut_v, lut_sp, code_v, out_v,
             g_sem, *, per_tile):
    cid = lax.axis_index("c")
    sid = lax.axis_index("s")
    wid = sid * NC + cid

    @pl.when(sid == 0)
    def _():
        pltpu.sync_copy(w0_hbm, w0_v)
        pltpu.sync_copy(w1_hbm, w1_v)
        pltpu.sync_copy(w2_hbm, w2_v)
        for i0 in range(D0):
            r0 = w0_v[i0, :]
            for i1 in range(D1):
                r01 = r0 + w1_v[i1, :]
                for i2 in range(D2):
                    lut_v[(i0 * D1 + i1) * D2 + i2, :] = r01 + w2_v[i2, :]
        pltpu.sync_copy(lut_v, lut_sp)
    plsc.subcore_barrier()

    base_w = wid * per_tile
    nchunks = per_tile // CHUNK

    def chunk_body(k, carry):
        base = base_w + k * CHUNK
        pltpu.sync_copy(codes_hbm.at[pl.ds(base, CHUNK)], code_v)
        descs = [
            pltpu.async_copy(lut_sp.at[code_v.at[pl.ds(j * GROW, GROW)]],
                             out_v.at[pl.ds(j * GROW, GROW)], g_sem)
            for j in range(NG)
        ]
        for d in descs:
            d.wait()
        pltpu.sync_copy(out_v, out_hbm.at[pl.ds(base, CHUNK), :])
        return carry

    lax.fori_loop(0, nchunks, chunk_body, 0)


def kernel(edge_attr, W0, W1, W2):
    E = edge_attr.shape[0]
    per_tile = E // NW
    assert per_tile * NW == E and per_tile % CHUNK == 0 and E % TC_BLK == 0, E
    a = edge_attr.astype(jnp.int32)

    codes = pl.pallas_call(
        _codes_body,
        grid=(E // TC_BLK,),
        in_specs=[pl.BlockSpec((TC_BLK, 3), lambda i: (i, 0))],
        out_specs=pl.BlockSpec((TC_BLK,), lambda i: (i,)),
        out_shape=jax.ShapeDtypeStruct((E,), jnp.int32),
    )(a)

    mesh = plsc.VectorSubcoreMesh(core_axis_name="c", subcore_axis_name="s",
                                  num_cores=NC, num_subcores=NS)
    return pl.kernel(
        functools.partial(_sc_body, per_tile=per_tile),
        out_type=jax.ShapeDtypeStruct((E, EMB), jnp.float32),
        mesh=mesh,
        compiler_params=pltpu.CompilerParams(needs_layout_passes=False,
                                             use_tc_tiling_on_sc=False),
        scratch_types=[
            pltpu.VMEM((D0, EMB), jnp.float32),
            pltpu.VMEM((D1, EMB), jnp.float32),
            pltpu.VMEM((D2, EMB), jnp.float32),
            pltpu.VMEM((NCODES, EMB), jnp.float32),
            pltpu.VMEM_SHARED((NCODES, EMB), jnp.float32),
            pltpu.VMEM((CHUNK,), jnp.int32),
            pltpu.VMEM((CHUNK, EMB), jnp.float32),
            pltpu.SemaphoreType.DMA,
        ],
    )(codes, W0, W1, W2)


# speedup vs baseline: 1.5987x; 1.5987x over previous
import functools

import jax
import jax.numpy as jnp
from jax import lax
from jax.experimental import pallas as pl
from jax.experimental.pallas import tpu as pltpu
from jax.experimental.pallas import tpu_sc as plsc

D0, D1, D2 = 5, 6, 2
EMB = 16
NCODES = D0 * D1 * D2
NC, NS, LANES = 2, 16, 16
NW = NC * NS
CHUNK = 2000
GROW = 80
NG = CHUNK // GROW


def _sc_body(codes_hbm, w0_hbm, w1_hbm, w2_hbm, out_hbm,
             w0_v, w1_v, w2_v, lut_v, lut_sp, code_v, out_v,
             g_sem, *, per_tile):
    cid = lax.axis_index("c")
    sid = lax.axis_index("s")
    wid = sid * NC + cid

    @pl.when(sid == 0)
    def _():
        pltpu.sync_copy(w0_hbm, w0_v)
        pltpu.sync_copy(w1_hbm, w1_v)
        pltpu.sync_copy(w2_hbm, w2_v)
        for i0 in range(D0):
            r0 = w0_v[i0, :]
            for i1 in range(D1):
                r01 = r0 + w1_v[i1, :]
                for i2 in range(D2):
                    lut_v[(i0 * D1 + i1) * D2 + i2, :] = r01 + w2_v[i2, :]
        pltpu.sync_copy(lut_v, lut_sp)
    plsc.subcore_barrier()

    base_w = wid * per_tile
    nchunks = per_tile // CHUNK

    def chunk_body(k, carry):
        base = base_w + k * CHUNK
        pltpu.sync_copy(codes_hbm.at[pl.ds(base, CHUNK)], code_v)
        descs = [
            pltpu.async_copy(lut_sp.at[code_v.at[pl.ds(j * GROW, GROW)]],
                             out_v.at[pl.ds(j * GROW, GROW)], g_sem)
            for j in range(NG)
        ]
        for d in descs:
            d.wait()
        pltpu.sync_copy(out_v, out_hbm.at[pl.ds(base, CHUNK), :])
        return carry

    lax.fori_loop(0, nchunks, chunk_body, 0)


def kernel(edge_attr, W0, W1, W2):
    E = edge_attr.shape[0]
    per_tile = E // NW
    assert per_tile * NW == E and per_tile % CHUNK == 0, E
    a = edge_attr.astype(jnp.int32)
    codes = (a[:, 0] * D1 + a[:, 1]) * D2 + a[:, 2]

    mesh = plsc.VectorSubcoreMesh(core_axis_name="c", subcore_axis_name="s",
                                  num_cores=NC, num_subcores=NS)
    out = pl.kernel(
        functools.partial(_sc_body, per_tile=per_tile),
        out_type=jax.ShapeDtypeStruct((E, EMB), jnp.float32),
        mesh=mesh,
        compiler_params=pltpu.CompilerParams(needs_layout_passes=False,
                                             use_tc_tiling_on_sc=False),
        scratch_types=[
            pltpu.VMEM((D0, EMB), jnp.float32),
            pltpu.VMEM((D1, EMB), jnp.float32),
            pltpu.VMEM((D2, EMB), jnp.float32),
            pltpu.VMEM((NCODES, EMB), jnp.float32),
            pltpu.VMEM_SHARED((NCODES, EMB), jnp.float32),
            pltpu.VMEM((CHUNK,), jnp.int32),
            pltpu.VMEM((CHUNK, EMB), jnp.float32),
            pltpu.SemaphoreType.DMA,
        ],
    )(codes, W0, W1, W2)
    one = W0[0, 0] * jnp.float32(0.0) + jnp.float32(1.0)
    return out * one


# speedup vs baseline: 7.9464x; 4.9706x over previous
import functools

import jax
import jax.numpy as jnp
from jax import lax
from jax.experimental import pallas as pl
from jax.experimental.pallas import tpu as pltpu
from jax.experimental.pallas import tpu_sc as plsc

D0, D1, D2 = 5, 6, 2
EMB = 16
NCODES = D0 * D1 * D2
LUTW = 64
NC, NS, LANES = 2, 16, 16
NW = NC * NS
TPC = 25
CHUNK = TPC * 128
TILE_W = 1024


def _sc_body(codes_hbm, w0_hbm, w1_hbm, w2_hbm, out_hbm,
             w0_v, w1_v, w2_v, lutT_v, code_v, buf0_v, buf1_v,
             *, n_chunks, plane_w):
    cid = lax.axis_index("c")
    sid = lax.axis_index("s")
    wid = sid * NC + cid

    pltpu.sync_copy(w0_hbm, w0_v)
    pltpu.sync_copy(w1_hbm, w1_v)
    pltpu.sync_copy(w2_hbm, w2_v)
    dim64 = lax.iota(jnp.int32, LANES) * LUTW
    for i0 in range(D0):
        r0 = w0_v[i0, :]
        for i1 in range(D1):
            r01 = r0 + w1_v[i1, :]
            for i2 in range(D2):
                code = (i0 * D1 + i1) * D2 + i2
                plsc.store_scatter(lutT_v, [dim64 + code], r01 + w2_v[i2, :])

    nch_w = jnp.where(wid < n_chunks % NW, n_chunks // NW + 1, n_chunks // NW)

    def chunk_body(i, carry):
        ch = wid + i * NW
        pltpu.sync_copy(codes_hbm.at[pl.ds(ch * CHUNK, CHUNK)], code_v)

        def tile_body(t, carry2):
            for j in range(8):
                cvec = code_v[pl.ds(t * 128 + j * 16, LANES)]
                for d in range(EMB):
                    val = plsc.load_gather(lutT_v, [cvec + d * LUTW])
                    buf = buf0_v if d < 8 else buf1_v
                    buf[pl.ds(t * TILE_W + (d % 8) * 128 + j * 16, LANES)] = val
            return carry2

        lax.fori_loop(0, TPC, tile_body, 0)
        pltpu.sync_copy(buf0_v, out_hbm.at[pl.ds(ch * TPC * TILE_W, TPC * TILE_W)])
        pltpu.sync_copy(buf1_v,
                        out_hbm.at[pl.ds(plane_w + ch * TPC * TILE_W, TPC * TILE_W)])
        return carry

    lax.fori_loop(0, nch_w, chunk_body, 0)


def kernel(edge_attr, W0, W1, W2):
    E = edge_attr.shape[0]
    assert E % (CHUNK) == 0, E
    n_chunks = E // CHUNK
    n_tiles = E // 128
    plane_w = n_tiles * TILE_W

    a = edge_attr.astype(jnp.int32)
    codes = (a[:, 0] * D1 + a[:, 1]) * D2 + a[:, 2]

    mesh = plsc.VectorSubcoreMesh(core_axis_name="c", subcore_axis_name="s",
                                  num_cores=NC, num_subcores=NS)
    flat = pl.kernel(
        functools.partial(_sc_body, n_chunks=n_chunks, plane_w=plane_w),
        out_type=jax.ShapeDtypeStruct((E * EMB,), jnp.float32),
        mesh=mesh,
        compiler_params=pltpu.CompilerParams(needs_layout_passes=False,
                                             use_tc_tiling_on_sc=False),
        scratch_types=[
            pltpu.VMEM((D0, EMB), jnp.float32),
            pltpu.VMEM((D1, EMB), jnp.float32),
            pltpu.VMEM((D2, EMB), jnp.float32),
            pltpu.VMEM((EMB * LUTW,), jnp.float32),
            pltpu.VMEM((CHUNK,), jnp.int32),
            pltpu.VMEM((TPC * TILE_W,), jnp.float32),
            pltpu.VMEM((TPC * TILE_W,), jnp.float32),
        ],
    )(codes, W0, W1, W2)
    arr = flat.reshape(2, n_tiles, 8, 128)
    return arr.transpose(1, 3, 0, 2).reshape(E, EMB)


# speedup vs baseline: 9.4948x; 1.1949x over previous
import functools

import jax
import jax.numpy as jnp
from jax import lax
from jax.experimental import pallas as pl
from jax.experimental.pallas import tpu as pltpu
from jax.experimental.pallas import tpu_sc as plsc

D0, D1, D2 = 5, 6, 2
EMB = 16
NCODES = D0 * D1 * D2
LUTW = 64
NC, NS, LANES = 2, 16, 16
NW = NC * NS
TPC = 25
CHUNK = TPC * 128
TILE_W = 1024
PLANE_B = TPC * TILE_W


def _sc_body(codes_hbm, w0_hbm, w1_hbm, w2_hbm, out_hbm,
             w0_v, w1_v, w2_v, lutT_v, code_v, buf0_v, buf1_v,
             in_sem, out_sem, *, n_chunks, plane_w):
    cid = lax.axis_index("c")
    sid = lax.axis_index("s")
    wid = sid * NC + cid

    pltpu.sync_copy(w0_hbm, w0_v)
    pltpu.sync_copy(w1_hbm, w1_v)
    pltpu.sync_copy(w2_hbm, w2_v)
    dim64 = lax.iota(jnp.int32, LANES) * LUTW
    for i0 in range(D0):
        r0 = w0_v[i0, :]
        for i1 in range(D1):
            r01 = r0 + w1_v[i1, :]
            for i2 in range(D2):
                code = (i0 * D1 + i1) * D2 + i2
                plsc.store_scatter(lutT_v, [dim64 + code], r01 + w2_v[i2, :])

    nch_w = jnp.where(wid < n_chunks % NW, n_chunks // NW + 1, n_chunks // NW)

    def fetch(i, slot):
        ch = wid + i * NW
        pltpu.async_copy(codes_hbm.at[pl.ds(ch * CHUNK, CHUNK)],
                         code_v.at[slot], in_sem)

    def drain_out(i, slot):
        ch = wid + i * NW
        pltpu.make_async_copy(
            buf0_v.at[slot], out_hbm.at[pl.ds(ch * PLANE_B, PLANE_B)],
            out_sem).wait()
        pltpu.make_async_copy(
            buf1_v.at[slot],
            out_hbm.at[pl.ds(plane_w + ch * PLANE_B, PLANE_B)],
            out_sem).wait()

    fetch(0, 0)

    def chunk_body(i, carry):
        slot = lax.rem(i, 2)
        ch = wid + i * NW

        @pl.when(i + 1 < nch_w)
        def _():
            fetch(i + 1, 1 - slot)

        pltpu.make_async_copy(codes_hbm.at[pl.ds(ch * CHUNK, CHUNK)],
                              code_v.at[slot], in_sem).wait()
        @pl.when(i >= 2)
        def _():
            drain_out(i - 2, slot)

        cv = code_v.at[slot]
        b0 = buf0_v.at[slot]
        b1 = buf1_v.at[slot]

        def tile_body(t, carry2):
            for j in range(8):
                cvec = cv[pl.ds(t * 128 + j * 16, LANES)]
                for d in range(EMB):
                    val = plsc.load_gather(lutT_v, [cvec + d * LUTW])
                    buf = b0 if d < 8 else b1
                    buf[pl.ds(t * TILE_W + (d % 8) * 128 + j * 16, LANES)] = val
            return carry2

        lax.fori_loop(0, TPC, tile_body, 0)

        pltpu.async_copy(b0, out_hbm.at[pl.ds(ch * PLANE_B, PLANE_B)], out_sem)
        pltpu.async_copy(b1, out_hbm.at[pl.ds(plane_w + ch * PLANE_B, PLANE_B)],
                         out_sem)
        return carry

    lax.fori_loop(0, nch_w, chunk_body, 0)

    @pl.when(nch_w >= 2)
    def _():
        drain_out(nch_w - 2, lax.rem(nch_w - 2, 2))
    drain_out(nch_w - 1, lax.rem(nch_w - 1, 2))


def kernel(edge_attr, W0, W1, W2):
    E = edge_attr.shape[0]
    assert E % CHUNK == 0, E
    n_chunks = E // CHUNK
    n_tiles = E // 128
    plane_w = n_tiles * TILE_W

    a = edge_attr.astype(jnp.int32)
    codes = (a[:, 0] * D1 + a[:, 1]) * D2 + a[:, 2]

    mesh = plsc.VectorSubcoreMesh(core_axis_name="c", subcore_axis_name="s",
                                  num_cores=NC, num_subcores=NS)
    flat = pl.kernel(
        functools.partial(_sc_body, n_chunks=n_chunks, plane_w=plane_w),
        out_type=jax.ShapeDtypeStruct((E * EMB,), jnp.float32),
        mesh=mesh,
        compiler_params=pltpu.CompilerParams(needs_layout_passes=False,
                                             use_tc_tiling_on_sc=False),
        scratch_types=[
            pltpu.VMEM((D0, EMB), jnp.float32),
            pltpu.VMEM((D1, EMB), jnp.float32),
            pltpu.VMEM((D2, EMB), jnp.float32),
            pltpu.VMEM((EMB * LUTW,), jnp.float32),
            pltpu.VMEM((2, CHUNK), jnp.int32),
            pltpu.VMEM((2, PLANE_B), jnp.float32),
            pltpu.VMEM((2, PLANE_B), jnp.float32),
            pltpu.SemaphoreType.DMA,
            pltpu.SemaphoreType.DMA,
        ],
    )(codes, W0, W1, W2)
    arr = flat.reshape(2, n_tiles, 8, 128)
    return arr.transpose(1, 3, 0, 2).reshape(E, EMB)
